# baseline (speedup 1.0000x reference)
import functools

import jax
import jax.numpy as jnp
from jax import lax
from jax.experimental import pallas as pl
from jax.experimental.pallas import tpu as pltpu
from jax.experimental.pallas import tpu_sc as plsc

N_ROWS = 8192
D = 4096
D1 = 1920
DSC = D - D1
N_IDX = 4096
NC = 2
NS = 16
NW = NC * NS
PER_SUB = N_IDX // NW
BINS_PER_SUB = N_ROWS // NS
CHUNK = 16
N_CHUNKS = PER_SUB // CHUNK
NBUF = 3
ROW_BLK = 512
LANES = 16


def _sc_hist(idx_hbm, s_hbm, w_hbm, den_hbm,
             idx_v, zidx_v, ssel_v, stage_v, shared, shared_d, sem):
    cid = lax.axis_index("c")
    sid = lax.axis_index("s")
    base = cid * (N_IDX // NC) + sid * PER_SUB

    def zero_chunk(k, _):
        stage_v[pl.ds(k * 16, 16)] = jnp.zeros((16,), jnp.float32)
        return 0
    lax.fori_loop(0, BINS_PER_SUB // 16, zero_chunk, 0)

    def zero_idx_chunk(k, _):
        zidx_v[pl.ds(k * 16, 16)] = jnp.zeros((16,), jnp.int32)
        return 0
    lax.fori_loop(0, PER_SUB // 16, zero_idx_chunk, 0)

    pltpu.sync_copy(stage_v, shared.at[pl.ds(sid * BINS_PER_SUB, BINS_PER_SUB)])
    pltpu.sync_copy(stage_v.at[pl.ds(0, 16)], shared_d)
    plsc.subcore_barrier()

    pltpu.sync_copy(idx_hbm.at[pl.ds(base, PER_SUB)], idx_v)
    pltpu.async_copy(s_hbm.at[idx_v], ssel_v, sem).wait()
    pltpu.sync_copy(ssel_v, shared.at[idx_v], add=True)
    pltpu.sync_copy(ssel_v, shared_d.at[zidx_v], add=True)
    plsc.subcore_barrier()

    pltpu.sync_copy(shared.at[pl.ds(sid * BINS_PER_SUB, BINS_PER_SUB)], stage_v)
    pltpu.sync_copy(stage_v, w_hbm.at[cid, pl.ds(sid * BINS_PER_SUB, BINS_PER_SUB)])

    @pl.when(sid == 0)
    def _pub_den():
        pltpu.sync_copy(shared_d, den_hbm.at[cid])


def _sc_pool(idx_hbm, s_hbm, tok_hbm, part_hbm,
             idx_v, ssel_v, g0, g1, g2, acc_v, sem0, sem1, sem2, sem_s):
    cid = lax.axis_index("c")
    sid = lax.axis_index("s")
    wid = sid * NC + cid
    base = wid * PER_SUB

    pltpu.sync_copy(idx_hbm.at[pl.ds(base, PER_SUB)], idx_v)
    pltpu.async_copy(s_hbm.at[idx_v], ssel_v, sem_s).wait()

    bufs = (g0, g1, g2)
    sems = (sem0, sem1, sem2)

    def start(c):
        return pltpu.async_copy(
            tok_hbm.at[idx_v.at[pl.ds(c * CHUNK, CHUNK)], pl.ds(D1, DSC)],
            bufs[c % NBUF], sems[c % NBUF])

    handles = [start(0), start(1)]

    for c in range(N_CHUNKS):
        handles[c].wait()
        if c + 2 < N_CHUNKS:
            handles.append(start(c + 2))
        g = bufs[c % NBUF]
        sv = ssel_v[pl.ds(c * CHUNK, LANES)]
        dnums = lax.GatherDimensionNumbers(
            offset_dims=(), collapsed_slice_dims=(0,), start_index_map=(0,))
        splats = [
            lax.gather(sv, jnp.full((LANES, 1), r, jnp.int32), dnums, (1,),
                       mode=lax.GatherScatterMode.PROMISE_IN_BOUNDS)
            for r in range(CHUNK)
        ]

        def wsum(k):
            p = [splats[r] * g[r, pl.ds(k * LANES, LANES)]
                 for r in range(CHUNK)]
            while len(p) > 1:
                p = [p[i] + p[i + 1] for i in range(0, len(p), 2)]
            return p[0]

        if c == 0:
            def kbody0(k, _):
                acc_v[pl.ds(k * LANES, LANES)] = wsum(k)
                return 0
            lax.fori_loop(0, DSC // LANES, kbody0, 0, unroll=8)
        else:
            def kbody(k, _):
                plsc.addupdate(acc_v.at[pl.ds(k * LANES, LANES)], wsum(k))
                return 0
            lax.fori_loop(0, DSC // LANES, kbody, 0, unroll=8)

    pltpu.sync_copy(acc_v, part_hbm.at[wid])


def _mv_body(w_ref, t_ref, o_ref):
    pid = pl.program_id(0)

    @pl.when(pid == 0)
    def _init():
        o_ref[...] = jnp.zeros_like(o_ref)

    wrow = w_ref[0, 0] + w_ref[1, 0]
    o_ref[...] += jax.lax.dot_general(
        wrow, t_ref[...], (((1,), (0,)), ((), ())),
        preferred_element_type=jnp.float32)


def _combine_body(left_ref, p_ref, den_ref, o_ref):
    denom = jnp.sum(den_ref[...]) + 1e-6
    o_ref[0:1, 0:D1] = left_ref[...] / denom
    o_ref[0:1, D1:D] = jnp.sum(p_ref[...], axis=0, keepdims=True) / denom


def kernel(tokens, s, idx):
    idx32 = idx.astype(jnp.int32)

    mesh = plsc.VectorSubcoreMesh(core_axis_name="c", subcore_axis_name="s")
    hist = functools.partial(
        pl.kernel,
        mesh=mesh,
        out_type=(
            jax.ShapeDtypeStruct((NC, N_ROWS), jnp.float32),
            jax.ShapeDtypeStruct((NC, 16), jnp.float32),
        ),
        scratch_types=[
            pltpu.VMEM((PER_SUB,), jnp.int32),
            pltpu.VMEM((PER_SUB,), jnp.int32),
            pltpu.VMEM((PER_SUB,), jnp.float32),
            pltpu.VMEM((BINS_PER_SUB,), jnp.float32),
            pltpu.VMEM_SHARED((N_ROWS,), jnp.float32),
            pltpu.VMEM_SHARED((16,), jnp.float32),
            pltpu.SemaphoreType.DMA,
        ],
    )(_sc_hist)
    w, den = hist(idx32, s)
    w4 = w.reshape(NC, N_ROWS // ROW_BLK, 1, ROW_BLK)

    pool = functools.partial(
        pl.kernel,
        mesh=mesh,
        cost_estimate=pl.CostEstimate(
            flops=2 * N_IDX * DSC, transcendentals=0,
            bytes_accessed=N_IDX * DSC * 4),
        out_type=jax.ShapeDtypeStruct((NW, DSC), jnp.float32),
        scratch_types=[
            pltpu.VMEM((PER_SUB,), jnp.int32),
            pltpu.VMEM((PER_SUB,), jnp.float32),
            pltpu.VMEM((CHUNK, DSC), jnp.float32),
            pltpu.VMEM((CHUNK, DSC), jnp.float32),
            pltpu.VMEM((CHUNK, DSC), jnp.float32),
            pltpu.VMEM((DSC,), jnp.float32),
            pltpu.SemaphoreType.DMA,
            pltpu.SemaphoreType.DMA,
            pltpu.SemaphoreType.DMA,
            pltpu.SemaphoreType.DMA,
        ],
    )(_sc_pool)
    part = pool(idx32, s, tokens)

    left = pl.pallas_call(
        _mv_body,
        grid=(N_ROWS // ROW_BLK,),
        in_specs=[
            pl.BlockSpec((NC, 1, 1, ROW_BLK), lambda i: (0, i, 0, 0)),
            pl.BlockSpec((ROW_BLK, D1), lambda i: (i, 0)),
        ],
        out_specs=pl.BlockSpec((1, D1), lambda i: (0, 0)),
        out_shape=jax.ShapeDtypeStruct((1, D1), jnp.float32),
    )(w4, tokens)

    out = pl.pallas_call(
        _combine_body,
        grid=(1,),
        in_specs=[
            pl.BlockSpec((1, D1), lambda i: (0, 0)),
            pl.BlockSpec((NW, DSC), lambda i: (0, 0)),
            pl.BlockSpec((NC, 16), lambda i: (0, 0)),
        ],
        out_specs=pl.BlockSpec((1, D), lambda i: (0, 0)),
        out_shape=jax.ShapeDtypeStruct((1, D), jnp.float32),
    )(left, part, den)

    return out


# speedup vs baseline: 1.0398x; 1.0398x over previous
import functools

import jax
import jax.numpy as jnp
from jax import lax
from jax.experimental import pallas as pl
from jax.experimental.pallas import tpu as pltpu
from jax.experimental.pallas import tpu_sc as plsc

N_ROWS = 8192
D = 4096
D1 = 2176
DSC = D - D1
N_IDX = 4096
NC = 2
NS = 16
NW = NC * NS
PER_SUB = N_IDX // NW
BINS_PER_SUB = N_ROWS // NS
CHUNK = 16
N_CHUNKS = PER_SUB // CHUNK
NBUF = 3
ROW_BLK = 512
LANES = 16


def _sc_hist(idx_hbm, s_hbm, w_hbm, den_hbm,
             idx_v, zidx_v, ssel_v, stage_v, shared, shared_d, sem):
    cid = lax.axis_index("c")
    sid = lax.axis_index("s")
    base = cid * (N_IDX // NC) + sid * PER_SUB

    def zero_chunk(k, _):
        stage_v[pl.ds(k * 16, 16)] = jnp.zeros((16,), jnp.float32)
        return 0
    lax.fori_loop(0, BINS_PER_SUB // 16, zero_chunk, 0)

    def zero_idx_chunk(k, _):
        zidx_v[pl.ds(k * 16, 16)] = jnp.zeros((16,), jnp.int32)
        return 0
    lax.fori_loop(0, PER_SUB // 16, zero_idx_chunk, 0)

    pltpu.sync_copy(stage_v, shared.at[pl.ds(sid * BINS_PER_SUB, BINS_PER_SUB)])
    pltpu.sync_copy(stage_v.at[pl.ds(0, 16)], shared_d)
    plsc.subcore_barrier()

    pltpu.sync_copy(idx_hbm.at[pl.ds(base, PER_SUB)], idx_v)
    pltpu.async_copy(s_hbm.at[idx_v], ssel_v, sem).wait()
    pltpu.sync_copy(ssel_v, shared.at[idx_v], add=True)
    pltpu.sync_copy(ssel_v, shared_d.at[zidx_v], add=True)
    plsc.subcore_barrier()

    pltpu.sync_copy(shared.at[pl.ds(sid * BINS_PER_SUB, BINS_PER_SUB)], stage_v)
    pltpu.sync_copy(stage_v, w_hbm.at[cid, pl.ds(sid * BINS_PER_SUB, BINS_PER_SUB)])

    @pl.when(sid == 0)
    def _pub_den():
        pltpu.sync_copy(shared_d, den_hbm.at[cid])


def _sc_pool(idx_hbm, s_hbm, tok_hbm, part_hbm,
             idx_v, ssel_v, g0, g1, g2, acc_v, sem0, sem1, sem2, sem_s):
    cid = lax.axis_index("c")
    sid = lax.axis_index("s")
    wid = sid * NC + cid
    base = wid * PER_SUB

    pltpu.sync_copy(idx_hbm.at[pl.ds(base, PER_SUB)], idx_v)
    pltpu.async_copy(s_hbm.at[idx_v], ssel_v, sem_s).wait()

    bufs = (g0, g1, g2)
    sems = (sem0, sem1, sem2)

    def start(c):
        return pltpu.async_copy(
            tok_hbm.at[idx_v.at[pl.ds(c * CHUNK, CHUNK)], pl.ds(D1, DSC)],
            bufs[c % NBUF], sems[c % NBUF])

    handles = [start(0), start(1)]

    for c in range(N_CHUNKS):
        handles[c].wait()
        if c + 2 < N_CHUNKS:
            handles.append(start(c + 2))
        g = bufs[c % NBUF]
        sv = ssel_v[pl.ds(c * CHUNK, LANES)]
        dnums = lax.GatherDimensionNumbers(
            offset_dims=(), collapsed_slice_dims=(0,), start_index_map=(0,))
        splats = [
            lax.gather(sv, jnp.full((LANES, 1), r, jnp.int32), dnums, (1,),
                       mode=lax.GatherScatterMode.PROMISE_IN_BOUNDS)
            for r in range(CHUNK)
        ]

        def wsum(k):
            p = [splats[r] * g[r, pl.ds(k * LANES, LANES)]
                 for r in range(CHUNK)]
            while len(p) > 1:
                p = [p[i] + p[i + 1] for i in range(0, len(p), 2)]
            return p[0]

        if c == 0:
            def kbody0(k, _):
                acc_v[pl.ds(k * LANES, LANES)] = wsum(k)
                return 0
            lax.fori_loop(0, DSC // LANES, kbody0, 0, unroll=8)
        else:
            def kbody(k, _):
                plsc.addupdate(acc_v.at[pl.ds(k * LANES, LANES)], wsum(k))
                return 0
            lax.fori_loop(0, DSC // LANES, kbody, 0, unroll=8)

    pltpu.sync_copy(acc_v, part_hbm.at[wid])


def _mv_body(w_ref, t_ref, o_ref):
    pid = pl.program_id(0)

    @pl.when(pid == 0)
    def _init():
        o_ref[...] = jnp.zeros_like(o_ref)

    wrow = w_ref[0, 0] + w_ref[1, 0]
    o_ref[...] += jax.lax.dot_general(
        wrow, t_ref[...], (((1,), (0,)), ((), ())),
        preferred_element_type=jnp.float32)


def _combine_body(left_ref, p_ref, den_ref, o_ref):
    denom = jnp.sum(den_ref[...]) + 1e-6
    o_ref[0:1, 0:D1] = left_ref[...] / denom
    o_ref[0:1, D1:D] = jnp.sum(p_ref[...], axis=0, keepdims=True) / denom


def kernel(tokens, s, idx):
    idx32 = idx.astype(jnp.int32)

    mesh = plsc.VectorSubcoreMesh(core_axis_name="c", subcore_axis_name="s")
    hist = functools.partial(
        pl.kernel,
        mesh=mesh,
        out_type=(
            jax.ShapeDtypeStruct((NC, N_ROWS), jnp.float32),
            jax.ShapeDtypeStruct((NC, 16), jnp.float32),
        ),
        scratch_types=[
            pltpu.VMEM((PER_SUB,), jnp.int32),
            pltpu.VMEM((PER_SUB,), jnp.int32),
            pltpu.VMEM((PER_SUB,), jnp.float32),
            pltpu.VMEM((BINS_PER_SUB,), jnp.float32),
            pltpu.VMEM_SHARED((N_ROWS,), jnp.float32),
            pltpu.VMEM_SHARED((16,), jnp.float32),
            pltpu.SemaphoreType.DMA,
        ],
    )(_sc_hist)
    w, den = hist(idx32, s)
    w4 = w.reshape(NC, N_ROWS // ROW_BLK, 1, ROW_BLK)

    pool = functools.partial(
        pl.kernel,
        mesh=mesh,
        cost_estimate=pl.CostEstimate(
            flops=2 * N_IDX * DSC, transcendentals=0,
            bytes_accessed=N_IDX * DSC * 4),
        out_type=jax.ShapeDtypeStruct((NW, DSC), jnp.float32),
        scratch_types=[
            pltpu.VMEM((PER_SUB,), jnp.int32),
            pltpu.VMEM((PER_SUB,), jnp.float32),
            pltpu.VMEM((CHUNK, DSC), jnp.float32),
            pltpu.VMEM((CHUNK, DSC), jnp.float32),
            pltpu.VMEM((CHUNK, DSC), jnp.float32),
            pltpu.VMEM((DSC,), jnp.float32),
            pltpu.SemaphoreType.DMA,
            pltpu.SemaphoreType.DMA,
            pltpu.SemaphoreType.DMA,
            pltpu.SemaphoreType.DMA,
        ],
    )(_sc_pool)
    part = pool(idx32, s, tokens)

    left = pl.pallas_call(
        _mv_body,
        grid=(N_ROWS // ROW_BLK,),
        in_specs=[
            pl.BlockSpec((NC, 1, 1, ROW_BLK), lambda i: (0, i, 0, 0)),
            pl.BlockSpec((ROW_BLK, D1), lambda i: (i, 0)),
        ],
        out_specs=pl.BlockSpec((1, D1), lambda i: (0, 0)),
        out_shape=jax.ShapeDtypeStruct((1, D1), jnp.float32),
    )(w4, tokens)

    out = pl.pallas_call(
        _combine_body,
        grid=(1,),
        in_specs=[
            pl.BlockSpec((1, D1), lambda i: (0, 0)),
            pl.BlockSpec((NW, DSC), lambda i: (0, 0)),
            pl.BlockSpec((NC, 16), lambda i: (0, 0)),
        ],
        out_specs=pl.BlockSpec((1, D), lambda i: (0, 0)),
        out_shape=jax.ShapeDtypeStruct((1, D), jnp.float32),
    )(left, part, den)

    return out


# speedup vs baseline: 1.0571x; 1.0166x over previous
import functools

import jax
import jax.numpy as jnp
from jax import lax
from jax.experimental import pallas as pl
from jax.experimental.pallas import tpu as pltpu
from jax.experimental.pallas import tpu_sc as plsc

N_ROWS = 8192
D = 4096
D1 = 2432
DSC = D - D1
N_IDX = 4096
NC = 2
NS = 16
NW = NC * NS
PER_SUB = N_IDX // NW
BINS_PER_SUB = N_ROWS // NS
CHUNK = 16
N_CHUNKS = PER_SUB // CHUNK
NBUF = 3
ROW_BLK = 512
LANES = 16


def _sc_hist(idx_hbm, s_hbm, w_hbm, den_hbm,
             idx_v, zidx_v, ssel_v, stage_v, shared, shared_d, sem):
    cid = lax.axis_index("c")
    sid = lax.axis_index("s")
    base = cid * (N_IDX // NC) + sid * PER_SUB

    def zero_chunk(k, _):
        stage_v[pl.ds(k * 16, 16)] = jnp.zeros((16,), jnp.float32)
        return 0
    lax.fori_loop(0, BINS_PER_SUB // 16, zero_chunk, 0)

    def zero_idx_chunk(k, _):
        zidx_v[pl.ds(k * 16, 16)] = jnp.zeros((16,), jnp.int32)
        return 0
    lax.fori_loop(0, PER_SUB // 16, zero_idx_chunk, 0)

    pltpu.sync_copy(stage_v, shared.at[pl.ds(sid * BINS_PER_SUB, BINS_PER_SUB)])
    pltpu.sync_copy(stage_v.at[pl.ds(0, 16)], shared_d)
    plsc.subcore_barrier()

    pltpu.sync_copy(idx_hbm.at[pl.ds(base, PER_SUB)], idx_v)
    pltpu.async_copy(s_hbm.at[idx_v], ssel_v, sem).wait()
    pltpu.sync_copy(ssel_v, shared.at[idx_v], add=True)
    pltpu.sync_copy(ssel_v, shared_d.at[zidx_v], add=True)
    plsc.subcore_barrier()

    pltpu.sync_copy(shared.at[pl.ds(sid * BINS_PER_SUB, BINS_PER_SUB)], stage_v)
    pltpu.sync_copy(stage_v, w_hbm.at[cid, pl.ds(sid * BINS_PER_SUB, BINS_PER_SUB)])

    @pl.when(sid == 0)
    def _pub_den():
        pltpu.sync_copy(shared_d, den_hbm.at[cid])


def _sc_pool(idx_hbm, s_hbm, tok_hbm, part_hbm,
             idx_v, ssel_v, g0, g1, g2, acc_v, sem0, sem1, sem2, sem_s):
    cid = lax.axis_index("c")
    sid = lax.axis_index("s")
    wid = sid * NC + cid
    base = wid * PER_SUB

    pltpu.sync_copy(idx_hbm.at[pl.ds(base, PER_SUB)], idx_v)
    pltpu.async_copy(s_hbm.at[idx_v], ssel_v, sem_s).wait()

    bufs = (g0, g1, g2)
    sems = (sem0, sem1, sem2)

    def start(c):
        return pltpu.async_copy(
            tok_hbm.at[idx_v.at[pl.ds(c * CHUNK, CHUNK)], pl.ds(D1, DSC)],
            bufs[c % NBUF], sems[c % NBUF])

    handles = [start(0), start(1)]

    for c in range(N_CHUNKS):
        handles[c].wait()
        if c + 2 < N_CHUNKS:
            handles.append(start(c + 2))
        g = bufs[c % NBUF]
        sv = ssel_v[pl.ds(c * CHUNK, LANES)]
        dnums = lax.GatherDimensionNumbers(
            offset_dims=(), collapsed_slice_dims=(0,), start_index_map=(0,))
        splats = [
            lax.gather(sv, jnp.full((LANES, 1), r, jnp.int32), dnums, (1,),
                       mode=lax.GatherScatterMode.PROMISE_IN_BOUNDS)
            for r in range(CHUNK)
        ]

        def wsum(k):
            p = [splats[r] * g[r, pl.ds(k * LANES, LANES)]
                 for r in range(CHUNK)]
            while len(p) > 1:
                p = [p[i] + p[i + 1] for i in range(0, len(p), 2)]
            return p[0]

        if c == 0:
            def kbody0(k, _):
                acc_v[pl.ds(k * LANES, LANES)] = wsum(k)
                return 0
            lax.fori_loop(0, DSC // LANES, kbody0, 0, unroll=8)
        else:
            def kbody(k, _):
                plsc.addupdate(acc_v.at[pl.ds(k * LANES, LANES)], wsum(k))
                return 0
            lax.fori_loop(0, DSC // LANES, kbody, 0, unroll=8)

    pltpu.sync_copy(acc_v, part_hbm.at[wid])


def _mv_body(w_ref, t_ref, o_ref):
    pid = pl.program_id(0)

    @pl.when(pid == 0)
    def _init():
        o_ref[...] = jnp.zeros_like(o_ref)

    wrow = w_ref[0, 0] + w_ref[1, 0]
    o_ref[...] += jax.lax.dot_general(
        wrow, t_ref[...], (((1,), (0,)), ((), ())),
        preferred_element_type=jnp.float32)


def _combine_body(left_ref, p_ref, den_ref, o_ref):
    denom = jnp.sum(den_ref[...]) + 1e-6
    o_ref[0:1, 0:D1] = left_ref[...] / denom
    o_ref[0:1, D1:D] = jnp.sum(p_ref[...], axis=0, keepdims=True) / denom


def kernel(tokens, s, idx):
    idx32 = idx.astype(jnp.int32)

    mesh = plsc.VectorSubcoreMesh(core_axis_name="c", subcore_axis_name="s")
    hist = functools.partial(
        pl.kernel,
        mesh=mesh,
        out_type=(
            jax.ShapeDtypeStruct((NC, N_ROWS), jnp.float32),
            jax.ShapeDtypeStruct((NC, 16), jnp.float32),
        ),
        scratch_types=[
            pltpu.VMEM((PER_SUB,), jnp.int32),
            pltpu.VMEM((PER_SUB,), jnp.int32),
            pltpu.VMEM((PER_SUB,), jnp.float32),
            pltpu.VMEM((BINS_PER_SUB,), jnp.float32),
            pltpu.VMEM_SHARED((N_ROWS,), jnp.float32),
            pltpu.VMEM_SHARED((16,), jnp.float32),
            pltpu.SemaphoreType.DMA,
        ],
    )(_sc_hist)
    w, den = hist(idx32, s)
    w4 = w.reshape(NC, N_ROWS // ROW_BLK, 1, ROW_BLK)

    pool = functools.partial(
        pl.kernel,
        mesh=mesh,
        cost_estimate=pl.CostEstimate(
            flops=2 * N_IDX * DSC, transcendentals=0,
            bytes_accessed=N_IDX * DSC * 4),
        out_type=jax.ShapeDtypeStruct((NW, DSC), jnp.float32),
        scratch_types=[
            pltpu.VMEM((PER_SUB,), jnp.int32),
            pltpu.VMEM((PER_SUB,), jnp.float32),
            pltpu.VMEM((CHUNK, DSC), jnp.float32),
            pltpu.VMEM((CHUNK, DSC), jnp.float32),
            pltpu.VMEM((CHUNK, DSC), jnp.float32),
            pltpu.VMEM((DSC,), jnp.float32),
            pltpu.SemaphoreType.DMA,
            pltpu.SemaphoreType.DMA,
            pltpu.SemaphoreType.DMA,
            pltpu.SemaphoreType.DMA,
        ],
    )(_sc_pool)
    part = pool(idx32, s, tokens)

    left = pl.pallas_call(
        _mv_body,
        grid=(N_ROWS // ROW_BLK,),
        in_specs=[
            pl.BlockSpec((NC, 1, 1, ROW_BLK), lambda i: (0, i, 0, 0)),
            pl.BlockSpec((ROW_BLK, D1), lambda i: (i, 0)),
        ],
        out_specs=pl.BlockSpec((1, D1), lambda i: (0, 0)),
        out_shape=jax.ShapeDtypeStruct((1, D1), jnp.float32),
    )(w4, tokens)

    out = pl.pallas_call(
        _combine_body,
        grid=(1,),
        in_specs=[
            pl.BlockSpec((1, D1), lambda i: (0, 0)),
            pl.BlockSpec((NW, DSC), lambda i: (0, 0)),
            pl.BlockSpec((NC, 16), lambda i: (0, 0)),
        ],
        out_specs=pl.BlockSpec((1, D), lambda i: (0, 0)),
        out_shape=jax.ShapeDtypeStruct((1, D), jnp.float32),
    )(left, part, den)

    return out
